# baseline (speedup 1.0000x reference)
import functools

import jax
import jax.numpy as jnp
from jax import lax
from jax.experimental import pallas as pl
from jax.experimental.pallas import tpu as pltpu
from jax.experimental.pallas import tpu_sc as plsc

_SMOOTH = 0.5
_BETA = 0.1
_MARGIN = 1.0
_SCALE = 0.1
_LANES = 128

_NW = 16
_SCL = 16
_ROW = 128


def _sep_loss(p):
    k = p.shape[0]
    pn2 = jnp.sum(p * p, axis=1, keepdims=True)
    norm = jnp.sqrt(pn2 + 1e-15)
    maxn = 1.0 - 1e-3
    q = p * jnp.where(norm > maxn, maxn / norm, 1.0)
    g = lax.dot_general(q, q, (((1,), (1,)), ((), ())),
                        preferred_element_type=jnp.float32)
    rows = lax.broadcasted_iota(jnp.int32, (k, k), 0)
    cols = lax.broadcasted_iota(jnp.int32, (k, k), 1)
    eye = (rows == cols).astype(jnp.float32)
    x2 = jnp.sum(g * eye, axis=1, keepdims=True)
    y2 = jnp.sum(g * eye, axis=0, keepdims=True)
    a = 1.0 - 2.0 * g + y2
    b = 1.0 - x2
    num2 = jnp.maximum(a * a * x2 + b * b * y2 - 2.0 * a * b * g, 0.0)
    den = jnp.maximum(1.0 - 2.0 * g + x2 * y2, 1e-15)
    ma2 = num2 / (den * den)
    arg = jnp.clip(jnp.sqrt(ma2 + 1e-15), 0.0, 1.0 - 1e-5)
    dist = jnp.log((1.0 + arg) / (1.0 - arg))
    viol = jnp.maximum(_MARGIN - dist, 0.0) * (1.0 - eye)
    return jnp.sum(viol) / (k * (k - 1))


def _lane_fold(x):
    r = x.shape[-1]
    part = x[..., 0:_LANES]
    for j in range(1, r // _LANES):
        part = part + x[..., j * _LANES:(j + 1) * _LANES]
    return part


def _tc_body(scores_ref, labels_ref, norms_ref, protos_ref,
             nll_ref, aux_ref, reg_acc):
    i = pl.program_id(0)
    nb = pl.num_programs(0)
    k, r = scores_ref.shape

    @pl.when(i == 0)
    def _init():
        reg_acc[...] = jnp.zeros_like(reg_acc)

    s = scores_ref[...] * _SCALE
    rs = jnp.sum(jnp.exp(s), axis=0, keepdims=True)
    lse = jnp.log(rs)
    lbl = labels_ref[0]
    oh = lax.broadcasted_iota(jnp.int32, s.shape, 0) == lbl
    picked = jnp.sum(jnp.where(oh, s, 0.0), axis=0, keepdims=True)
    nll_ref[...] = jnp.reshape(lse - picked, (1, 1, r))
    nrm = norms_ref[0]
    reg_acc[...] += _lane_fold(nrm * nrm)

    @pl.when(i == nb - 1)
    def _fini():
        reg = jnp.sum(reg_acc[...]) / (nb * r)
        p = jnp.transpose(protos_ref[...])
        aux = _BETA * reg + _sep_loss(p)
        aux_ref[...] = jnp.full((1, _LANES), aux, jnp.float32)


def _newton_sqrt(y):
    x = y
    for _ in range(16):
        x = 0.5 * (x + y / x)
    return x


def _sc_body(k, ch, labels_hbm, nll_hbm, aux_hbm, out_hbm,
             lbl_v, nll_v, tab_cnt, tab_nll, tab_cnt2, tab_nll2,
             res_v, red_v, aux_v, shared_sp):
    wid = lax.axis_index("s") + lax.axis_index("c") * _NW
    base = wid * ch
    pltpu.sync_copy(labels_hbm.at[pl.ds(base, ch)], lbl_v)
    pltpu.sync_copy(nll_hbm.at[pl.ds(base, ch)], nll_v)

    zeros16 = jnp.zeros((_SCL,), jnp.float32)
    ones16 = jnp.ones((_SCL,), jnp.float32)

    def _zero(j, carry):
        for row in range(_SCL):
            tab_cnt[row, pl.ds(j * _SCL, _SCL)] = zeros16
            tab_nll[row, pl.ds(j * _SCL, _SCL)] = zeros16
            tab_cnt2[row, pl.ds(j * _SCL, _SCL)] = zeros16
            tab_nll2[row, pl.ds(j * _SCL, _SCL)] = zeros16
        return carry

    lax.fori_loop(0, _ROW // _SCL, _zero, 0)

    lane_ids = lax.iota(jnp.int32, _SCL)
    unroll = 8

    def _accum(j, carry):
        for u in range(unroll):
            off = j * (_SCL * unroll) + u * _SCL
            lb = lbl_v[pl.ds(off, _SCL)]
            nv = nll_v[pl.ds(off, _SCL)]
            if u % 2 == 0:
                plsc.addupdate_scatter(tab_cnt, [lane_ids, lb], ones16)
                plsc.addupdate_scatter(tab_nll, [lane_ids, lb], nv)
            else:
                plsc.addupdate_scatter(tab_cnt2, [lane_ids, lb], ones16)
                plsc.addupdate_scatter(tab_nll2, [lane_ids, lb], nv)
        return carry

    lax.fori_loop(0, ch // (_SCL * unroll), _accum, 0)

    for c in range(k // _SCL):
        acc_c = tab_cnt[0, pl.ds(c * _SCL, _SCL)]
        acc_n = tab_nll[0, pl.ds(c * _SCL, _SCL)]
        for row in range(_SCL):
            if row > 0:
                acc_c = acc_c + tab_cnt[row, pl.ds(c * _SCL, _SCL)]
                acc_n = acc_n + tab_nll[row, pl.ds(c * _SCL, _SCL)]
            acc_c = acc_c + tab_cnt2[row, pl.ds(c * _SCL, _SCL)]
            acc_n = acc_n + tab_nll2[row, pl.ds(c * _SCL, _SCL)]
        res_v[pl.ds(c * _SCL, _SCL)] = acc_c
        res_v[pl.ds(k + c * _SCL, _SCL)] = acc_n

    pltpu.sync_copy(res_v, shared_sp.at[pl.ds(wid * 2 * k, 2 * k)])
    plsc.subcore_barrier()

    @pl.when(wid == 0)
    def _combine():
        pltpu.sync_copy(shared_sp, red_v)
        pltpu.sync_copy(aux_hbm.at[pl.ds(0, _SCL)], aux_v)
        nch = k // _SCL
        cnts = []
        nlls = []
        for c in range(nch):
            acc_c = red_v[pl.ds(c * _SCL, _SCL)]
            acc_n = red_v[pl.ds(k + c * _SCL, _SCL)]
            for w in range(1, _NW):
                acc_c = acc_c + red_v[pl.ds(w * 2 * k + c * _SCL, _SCL)]
                acc_n = acc_n + red_v[pl.ds(w * 2 * k + k + c * _SCL, _SCL)]
            cnts.append(acc_c)
            nlls.append(acc_n)
        maxv = cnts[0]
        for c in range(1, nch):
            maxv = jnp.maximum(maxv, cnts[c])
        maxc = jnp.max(maxv)
        num = jnp.zeros((_SCL,), jnp.float32)
        den = jnp.zeros((_SCL,), jnp.float32)
        for c in range(nch):
            clamped = jnp.maximum(cnts[c], 1.0)
            w_c = _newton_sqrt((maxc * jnp.ones((_SCL,), jnp.float32))
                               / clamped)
            num = num + w_c * nlls[c]
            den = den + w_c * cnts[c]
        num_s = jnp.sum(num) * jnp.ones((_SCL,), jnp.float32)
        den_s = jnp.sum(den) * jnp.ones((_SCL,), jnp.float32)
        total = num_s / den_s + aux_v[...]
        res_v[pl.ds(0, _SCL)] = total
        pltpu.sync_copy(res_v.at[pl.ds(0, _SCL)], out_hbm)


def kernel(embeddings, scores, labels, prototypes, pre_expmap_norms):
    del embeddings
    m, k = scores.shape
    d = prototypes.shape[1]
    r = 16384
    nb = m // r
    scores_t = scores.T
    protos_t = prototypes.T
    lbl_i32 = labels.astype(jnp.int32)
    lbl3 = lbl_i32.reshape(nb, 1, r)
    nrm3 = pre_expmap_norms.reshape(nb, 1, r)
    nll3, aux = pl.pallas_call(
        _tc_body,
        grid=(nb,),
        in_specs=[
            pl.BlockSpec((k, r), lambda i: (0, i)),
            pl.BlockSpec((1, 1, r), lambda i: (i, 0, 0)),
            pl.BlockSpec((1, 1, r), lambda i: (i, 0, 0)),
            pl.BlockSpec((d, k), lambda i: (0, 0)),
        ],
        out_specs=[
            pl.BlockSpec((1, 1, r), lambda i: (i, 0, 0)),
            pl.BlockSpec((1, _LANES), lambda i: (0, 0)),
        ],
        out_shape=[
            jax.ShapeDtypeStruct((nb, 1, r), jnp.float32),
            jax.ShapeDtypeStruct((1, _LANES), jnp.float32),
        ],
        scratch_shapes=[
            pltpu.VMEM((1, _LANES), jnp.float32),
        ],
        compiler_params=pltpu.CompilerParams(
            dimension_semantics=("arbitrary",)),
    )(scores_t, lbl3, nrm3, protos_t)

    ch = m // _NW
    sc_kernel = functools.partial(
        pl.kernel,
        out_type=jax.ShapeDtypeStruct((_SCL,), jnp.float32),
        mesh=plsc.VectorSubcoreMesh(
            core_axis_name="c", subcore_axis_name="s", num_cores=1),
        compiler_params=pltpu.CompilerParams(needs_layout_passes=False),
        scratch_types=[
            pltpu.VMEM((ch,), jnp.int32),
            pltpu.VMEM((ch,), jnp.float32),
            pltpu.VMEM((_SCL, _ROW), jnp.float32),
            pltpu.VMEM((_SCL, _ROW), jnp.float32),
            pltpu.VMEM((_SCL, _ROW), jnp.float32),
            pltpu.VMEM((_SCL, _ROW), jnp.float32),
            pltpu.VMEM((2 * k,), jnp.float32),
            pltpu.VMEM((_NW * 2 * k,), jnp.float32),
            pltpu.VMEM((_SCL,), jnp.float32),
            pltpu.VMEM_SHARED((_NW * 2 * k,), jnp.float32),
        ],
    )(functools.partial(_sc_body, k, ch))
    out = sc_kernel(lbl_i32, nll3.reshape(m), aux.reshape(_LANES))
    return out[0]


# speedup vs baseline: 1.1423x; 1.1423x over previous
import functools

import jax
import jax.numpy as jnp
from jax import lax
from jax.experimental import pallas as pl
from jax.experimental.pallas import tpu as pltpu
from jax.experimental.pallas import tpu_sc as plsc

_SMOOTH = 0.5
_BETA = 0.1
_MARGIN = 1.0
_SCALE = 0.1
_LANES = 128

_NW = 16
_SCL = 16
_ROW = 128


def _sep_loss(p):
    k = p.shape[0]
    pn2 = jnp.sum(p * p, axis=1, keepdims=True)
    norm = jnp.sqrt(pn2 + 1e-15)
    maxn = 1.0 - 1e-3
    q = p * jnp.where(norm > maxn, maxn / norm, 1.0)
    g = lax.dot_general(q, q, (((1,), (1,)), ((), ())),
                        preferred_element_type=jnp.float32)
    rows = lax.broadcasted_iota(jnp.int32, (k, k), 0)
    cols = lax.broadcasted_iota(jnp.int32, (k, k), 1)
    eye = (rows == cols).astype(jnp.float32)
    x2 = jnp.sum(g * eye, axis=1, keepdims=True)
    y2 = jnp.sum(g * eye, axis=0, keepdims=True)
    a = 1.0 - 2.0 * g + y2
    b = 1.0 - x2
    num2 = jnp.maximum(a * a * x2 + b * b * y2 - 2.0 * a * b * g, 0.0)
    den = jnp.maximum(1.0 - 2.0 * g + x2 * y2, 1e-15)
    ma2 = num2 / (den * den)
    arg = jnp.clip(jnp.sqrt(ma2 + 1e-15), 0.0, 1.0 - 1e-5)
    dist = jnp.log((1.0 + arg) / (1.0 - arg))
    viol = jnp.maximum(_MARGIN - dist, 0.0) * (1.0 - eye)
    return jnp.sum(viol) / (k * (k - 1))


def _lane_fold(x):
    r = x.shape[-1]
    part = x[..., 0:_LANES]
    for j in range(1, r // _LANES):
        part = part + x[..., j * _LANES:(j + 1) * _LANES]
    return part


def _tc_body(scores_ref, labels_ref, norms_ref, protos_ref,
             stats_ref, cnt_acc, nll_acc, reg_acc):
    i = pl.program_id(0)
    nb = pl.num_programs(0)
    k, r = scores_ref.shape

    @pl.when(i == 0)
    def _init():
        cnt_acc[...] = jnp.zeros_like(cnt_acc)
        nll_acc[...] = jnp.zeros_like(nll_acc)
        reg_acc[...] = jnp.zeros_like(reg_acc)

    s = scores_ref[...] * _SCALE
    rs = jnp.sum(jnp.exp(s), axis=0, keepdims=True)
    lse = jnp.log(rs)
    lbl = labels_ref[0]
    oh = lax.broadcasted_iota(jnp.int32, s.shape, 0) == lbl
    cnt_acc[...] += _lane_fold(oh.astype(jnp.float32))
    nll_acc[...] += _lane_fold(jnp.where(oh, lse - s, 0.0))
    nrm = norms_ref[0]
    reg_acc[...] += _lane_fold(nrm * nrm)

    @pl.when(i == nb - 1)
    def _fini():
        cnt_t = jnp.transpose(
            jnp.sum(cnt_acc[...], axis=1, keepdims=True))
        nll_t = jnp.transpose(
            jnp.sum(nll_acc[...], axis=1, keepdims=True))
        reg = jnp.sum(reg_acc[...]) / (nb * r)
        p = jnp.transpose(protos_ref[...])
        aux = _BETA * reg + _sep_loss(p)
        aux_t = jnp.full((1, k), aux, jnp.float32)
        stats_ref[...] = jnp.concatenate([cnt_t, nll_t, aux_t], axis=0)


def _newton_sqrt(y):
    x = y
    for _ in range(16):
        x = 0.5 * (x + y / x)
    return x


def _sc_body(k, ch, labels_hbm, w_hbm,
             lbl_v, tab_cnt, res_v, red_v, shared_sp):
    wid = lax.axis_index("s") + lax.axis_index("c") * _NW
    base = wid * ch
    pltpu.sync_copy(labels_hbm.at[pl.ds(base, ch)], lbl_v)

    zeros16 = jnp.zeros((_SCL,), jnp.float32)
    ones16 = jnp.ones((_SCL,), jnp.float32)

    def _zero(j, carry):
        for row in range(_SCL):
            tab_cnt[row, pl.ds(j * _SCL, _SCL)] = zeros16
        return carry

    lax.fori_loop(0, _ROW // _SCL, _zero, 0)

    lane_ids = lax.iota(jnp.int32, _SCL)
    unroll = 8

    def _accum(j, carry):
        for u in range(unroll):
            off = j * (_SCL * unroll) + u * _SCL
            lb = lbl_v[pl.ds(off, _SCL)]
            plsc.addupdate_scatter(tab_cnt, [lane_ids, lb], ones16)
        return carry

    lax.fori_loop(0, ch // (_SCL * unroll), _accum, 0)

    for c in range(k // _SCL):
        acc_c = tab_cnt[0, pl.ds(c * _SCL, _SCL)]
        for row in range(1, _SCL):
            acc_c = acc_c + tab_cnt[row, pl.ds(c * _SCL, _SCL)]
        res_v[pl.ds(c * _SCL, _SCL)] = acc_c

    pltpu.sync_copy(res_v.at[pl.ds(0, k)], shared_sp.at[pl.ds(wid * k, k)])
    plsc.subcore_barrier()

    @pl.when(wid == 0)
    def _combine():
        pltpu.sync_copy(shared_sp, red_v)
        nch = k // _SCL
        cnts = []
        for c in range(nch):
            acc_c = red_v[pl.ds(c * _SCL, _SCL)]
            for w in range(1, _NW):
                acc_c = acc_c + red_v[pl.ds(w * k + c * _SCL, _SCL)]
            cnts.append(jnp.maximum(acc_c, 1.0))
        maxv = cnts[0]
        for c in range(1, nch):
            maxv = jnp.maximum(maxv, cnts[c])
        maxc = jnp.max(maxv)
        ones = jnp.ones((_SCL,), jnp.float32)
        for c in range(nch):
            res_v[pl.ds(c * _SCL, _SCL)] = _newton_sqrt(
                (maxc * ones) / cnts[c])
        pltpu.sync_copy(res_v.at[pl.ds(0, k)], w_hbm)


def _join_body(stats_ref, w_ref, out_ref):
    stats = stats_ref[...]
    w = w_ref[...]
    num = jnp.sum(w * stats[1:2, :])
    den = jnp.sum(w * stats[0:1, :])
    out_ref[...] = jnp.reshape(num / den + stats[2, 0], (1, 1))


def kernel(embeddings, scores, labels, prototypes, pre_expmap_norms):
    del embeddings
    m, k = scores.shape
    d = prototypes.shape[1]
    r = 16384
    nb = m // r
    scores_t = scores.T
    protos_t = prototypes.T
    lbl_i32 = labels.astype(jnp.int32)
    lbl3 = lbl_i32.reshape(nb, 1, r)
    nrm3 = pre_expmap_norms.reshape(nb, 1, r)

    ch = m // _NW
    w_vec = functools.partial(
        pl.kernel,
        out_type=jax.ShapeDtypeStruct((k,), jnp.float32),
        mesh=plsc.VectorSubcoreMesh(
            core_axis_name="c", subcore_axis_name="s", num_cores=1),
        compiler_params=pltpu.CompilerParams(needs_layout_passes=False),
        scratch_types=[
            pltpu.VMEM((ch,), jnp.int32),
            pltpu.VMEM((_SCL, _ROW), jnp.float32),
            pltpu.VMEM((_ROW,), jnp.float32),
            pltpu.VMEM((_NW * k,), jnp.float32),
            pltpu.VMEM_SHARED((_NW * k,), jnp.float32),
        ],
    )(functools.partial(_sc_body, k, ch))(lbl_i32)

    stats = pl.pallas_call(
        _tc_body,
        grid=(nb,),
        in_specs=[
            pl.BlockSpec((k, r), lambda i: (0, i)),
            pl.BlockSpec((1, 1, r), lambda i: (i, 0, 0)),
            pl.BlockSpec((1, 1, r), lambda i: (i, 0, 0)),
            pl.BlockSpec((d, k), lambda i: (0, 0)),
        ],
        out_specs=pl.BlockSpec((3, k), lambda i: (0, 0)),
        out_shape=jax.ShapeDtypeStruct((3, k), jnp.float32),
        scratch_shapes=[
            pltpu.VMEM((k, _LANES), jnp.float32),
            pltpu.VMEM((k, _LANES), jnp.float32),
            pltpu.VMEM((1, _LANES), jnp.float32),
        ],
        compiler_params=pltpu.CompilerParams(
            dimension_semantics=("arbitrary",)),
    )(scores_t, lbl3, nrm3, protos_t)

    out = pl.pallas_call(
        _join_body,
        out_shape=jax.ShapeDtypeStruct((1, 1), jnp.float32),
    )(stats, w_vec.reshape(1, k))
    return out[0, 0]
